# baseline (speedup 1.0000x reference)
import jax
import jax.numpy as jnp
from jax.experimental import pallas as pl

_BATCH = 16384
_DIM = 256
_BLOCK = 1024


def _rowdot_kernel(gu_ref, gi_ref, out_ref):
    out_ref[:] = jnp.sum(gu_ref[:] * gi_ref[:], axis=1)


def kernel(gu, gi):
    xui = pl.pallas_call(
        _rowdot_kernel,
        grid=(_BATCH // _BLOCK,),
        in_specs=[
            pl.BlockSpec((_BLOCK, _DIM), lambda i: (i, 0)),
            pl.BlockSpec((_BLOCK, _DIM), lambda i: (i, 0)),
        ],
        out_specs=pl.BlockSpec((_BLOCK,), lambda i: (i,)),
        out_shape=jax.ShapeDtypeStruct((_BATCH,), jnp.float32),
    )(gu, gi)
    return (xui, gu, gi)


# speedup vs baseline: 1.5251x; 1.5251x over previous
import jax
import jax.numpy as jnp
from jax.experimental import pallas as pl

_BATCH = 16384
_DIM = 256
_BLOCK = 1024


def _rowdot_copy_kernel(gu_ref, gi_ref, xui_ref, gu_out_ref, gi_out_ref):
    gu = gu_ref[:]
    gi = gi_ref[:]
    xui_ref[:] = jnp.sum(gu * gi, axis=1)
    gu_out_ref[:] = gu
    gi_out_ref[:] = gi


def kernel(gu, gi):
    xui, gu_out, gi_out = pl.pallas_call(
        _rowdot_copy_kernel,
        grid=(_BATCH // _BLOCK,),
        in_specs=[
            pl.BlockSpec((_BLOCK, _DIM), lambda i: (i, 0)),
            pl.BlockSpec((_BLOCK, _DIM), lambda i: (i, 0)),
        ],
        out_specs=[
            pl.BlockSpec((_BLOCK,), lambda i: (i,)),
            pl.BlockSpec((_BLOCK, _DIM), lambda i: (i, 0)),
            pl.BlockSpec((_BLOCK, _DIM), lambda i: (i, 0)),
        ],
        out_shape=[
            jax.ShapeDtypeStruct((_BATCH,), jnp.float32),
            jax.ShapeDtypeStruct((_BATCH, _DIM), jnp.float32),
            jax.ShapeDtypeStruct((_BATCH, _DIM), jnp.float32),
        ],
    )(gu, gi)
    return (xui, gu_out, gi_out)


# speedup vs baseline: 1.7693x; 1.1601x over previous
import jax
import jax.numpy as jnp
from jax.experimental import pallas as pl

_BATCH = 16384
_DIM = 256
_BLOCK = 2048


def _rowdot_copy_kernel(gu_ref, gi_ref, xui_ref, gu_out_ref, gi_out_ref):
    gu = gu_ref[:]
    gi = gi_ref[:]
    xui_ref[:] = jnp.sum(gu * gi, axis=1)
    gu_out_ref[:] = gu
    gi_out_ref[:] = gi


def kernel(gu, gi):
    xui, gu_out, gi_out = pl.pallas_call(
        _rowdot_copy_kernel,
        grid=(_BATCH // _BLOCK,),
        in_specs=[
            pl.BlockSpec((_BLOCK, _DIM), lambda i: (i, 0)),
            pl.BlockSpec((_BLOCK, _DIM), lambda i: (i, 0)),
        ],
        out_specs=[
            pl.BlockSpec((_BLOCK,), lambda i: (i,)),
            pl.BlockSpec((_BLOCK, _DIM), lambda i: (i, 0)),
            pl.BlockSpec((_BLOCK, _DIM), lambda i: (i, 0)),
        ],
        out_shape=[
            jax.ShapeDtypeStruct((_BATCH,), jnp.float32),
            jax.ShapeDtypeStruct((_BATCH, _DIM), jnp.float32),
            jax.ShapeDtypeStruct((_BATCH, _DIM), jnp.float32),
        ],
    )(gu, gi)
    return (xui, gu_out, gi_out)


# speedup vs baseline: 1.8008x; 1.0178x over previous
import jax
import jax.numpy as jnp
from jax.experimental import pallas as pl

_BATCH = 16384
_DIM = 256
_BLOCK = 4096


def _rowdot_copy_kernel(gu_ref, gi_ref, xui_ref, gu_out_ref, gi_out_ref):
    gu = gu_ref[:]
    gi = gi_ref[:]
    xui_ref[:] = jnp.sum(gu * gi, axis=1)
    gu_out_ref[:] = gu
    gi_out_ref[:] = gi


def kernel(gu, gi):
    xui, gu_out, gi_out = pl.pallas_call(
        _rowdot_copy_kernel,
        grid=(_BATCH // _BLOCK,),
        in_specs=[
            pl.BlockSpec((_BLOCK, _DIM), lambda i: (i, 0)),
            pl.BlockSpec((_BLOCK, _DIM), lambda i: (i, 0)),
        ],
        out_specs=[
            pl.BlockSpec((_BLOCK,), lambda i: (i,)),
            pl.BlockSpec((_BLOCK, _DIM), lambda i: (i, 0)),
            pl.BlockSpec((_BLOCK, _DIM), lambda i: (i, 0)),
        ],
        out_shape=[
            jax.ShapeDtypeStruct((_BATCH,), jnp.float32),
            jax.ShapeDtypeStruct((_BATCH, _DIM), jnp.float32),
            jax.ShapeDtypeStruct((_BATCH, _DIM), jnp.float32),
        ],
    )(gu, gi)
    return (xui, gu_out, gi_out)
